# Initial kernel scaffold; baseline (speedup 1.0000x reference)
import jax
import jax.numpy as jnp
from jax.experimental import pallas as pl


def kernel(x, token_table, pos_table):
    raise NotImplementedError("write your pallas kernel here")


# speedup vs baseline: 2.1707x; 2.1707x over previous
import functools

import jax
import jax.numpy as jnp
from jax import lax
from jax.experimental import pallas as pl
from jax.experimental.pallas import tpu as pltpu
from jax.experimental.pallas import tpu_sc as plsc

BATCH = 4096
MAXLEN = 200
EMB = 64

NUM_CORES = 2
NUM_SUBCORES = 16
NUM_WORKERS = NUM_CORES * NUM_SUBCORES
TOTAL_ROWS = BATCH * MAXLEN
ROWS_PER_WORKER = TOTAL_ROWS // NUM_WORKERS
CHUNK = 128
CHUNKS_PER_WORKER = ROWS_PER_WORKER // CHUNK
LANES = 16
VECS_PER_ROW = EMB // LANES


def _sc_body(x2_hbm, tab_hbm, pos_hbm, out_hbm, idx_v, posrep_v, rows_v, sem):
    c = lax.axis_index("c")
    s = lax.axis_index("s")
    wid = s * NUM_CORES + c

    pltpu.sync_copy(pos_hbm, posrep_v.at[pl.ds(0, MAXLEN)])
    pltpu.sync_copy(pos_hbm.at[pl.ds(0, CHUNK)],
                    posrep_v.at[pl.ds(MAXLEN, CHUNK)])

    pltpu.sync_copy(x2_hbm.at[pl.ds(wid * CHUNKS_PER_WORKER, CHUNKS_PER_WORKER)],
                    idx_v)

    def chunk_body(g, carry):
        pltpu.async_copy(tab_hbm.at[idx_v.at[g]], rows_v, sem).wait()
        phase = lax.rem(g * CHUNK, MAXLEN)

        def row_body(i, c2):
            for j in range(VECS_PER_ROW):
                sl = pl.ds(j * LANES, LANES)
                rows_v[i, sl] = rows_v[i, sl] + posrep_v[phase + i, sl]
            return c2

        lax.fori_loop(0, CHUNK, row_body, 0)
        pltpu.sync_copy(
            rows_v,
            out_hbm.at[pl.ds(wid * ROWS_PER_WORKER + g * CHUNK, CHUNK)])
        return carry

    lax.fori_loop(0, CHUNKS_PER_WORKER, chunk_body, 0)


@jax.jit
def kernel(x, token_table, pos_table):
    x2 = x.reshape(NUM_WORKERS * CHUNKS_PER_WORKER, CHUNK)
    mesh = plsc.VectorSubcoreMesh(core_axis_name="c", subcore_axis_name="s")
    out = pl.kernel(
        _sc_body,
        out_type=jax.ShapeDtypeStruct((TOTAL_ROWS, EMB), jnp.float32),
        mesh=mesh,
        scratch_types=[
            pltpu.VMEM((CHUNKS_PER_WORKER, CHUNK), jnp.int32),
            pltpu.VMEM((MAXLEN + CHUNK, EMB), jnp.float32),
            pltpu.VMEM((CHUNK, EMB), jnp.float32),
            pltpu.SemaphoreType.DMA,
        ],
        compiler_params=pltpu.CompilerParams(use_tc_tiling_on_sc=False),
    )(x2, token_table, pos_table)
    return out.reshape(BATCH, MAXLEN, EMB)


# speedup vs baseline: 2.5550x; 1.1770x over previous
import functools

import jax
import jax.numpy as jnp
from jax import lax
from jax.experimental import pallas as pl
from jax.experimental.pallas import tpu as pltpu
from jax.experimental.pallas import tpu_sc as plsc

BATCH = 4096
MAXLEN = 200
EMB = 64

NUM_CORES = 2
NUM_SUBCORES = 16
NUM_WORKERS = NUM_CORES * NUM_SUBCORES
TOTAL_ROWS = BATCH * MAXLEN
ROWS_PER_WORKER = TOTAL_ROWS // NUM_WORKERS
CHUNK = 128
CHUNKS_PER_WORKER = ROWS_PER_WORKER // CHUNK
LANES = 16
VECS_PER_ROW = EMB // LANES


def _sc_body(x2_hbm, tab_hbm, pos_hbm, out_hbm, idx_v, posrep_v,
             rows_a, rows_b, sg_a, sg_b, ss_a, ss_b):
    c = lax.axis_index("c")
    s = lax.axis_index("s")
    wid = s * NUM_CORES + c

    rows = (rows_a, rows_b)
    sg = (sg_a, sg_b)
    ss = (ss_a, ss_b)

    pltpu.sync_copy(pos_hbm, posrep_v.at[pl.ds(0, MAXLEN)])
    pltpu.sync_copy(pos_hbm.at[pl.ds(0, CHUNK)],
                    posrep_v.at[pl.ds(MAXLEN, CHUNK)])

    pltpu.sync_copy(x2_hbm.at[pl.ds(wid * CHUNKS_PER_WORKER, CHUNKS_PER_WORKER)],
                    idx_v)

    def out_slice(g):
        return out_hbm.at[pl.ds(wid * ROWS_PER_WORKER + g * CHUNK, CHUNK)]

    def start_gather(g, b):
        pltpu.async_copy(tab_hbm.at[idx_v.at[g]], rows[b], sg[b])

    def wait_gather(g, b):
        pltpu.make_async_copy(tab_hbm.at[idx_v.at[g]], rows[b], sg[b]).wait()

    def start_scatter(g, b):
        pltpu.async_copy(rows[b], out_slice(g), ss[b])

    def wait_scatter(g, b):
        pltpu.make_async_copy(rows[b], out_slice(g), ss[b]).wait()

    def add_pos(g, b):
        buf = rows[b]
        phase = lax.rem(g * CHUNK, MAXLEN)

        def row_body(i, c2):
            for j in range(VECS_PER_ROW):
                sl = pl.ds(j * LANES, LANES)
                buf[i, sl] = buf[i, sl] + posrep_v[phase + i, sl]
            return c2

        lax.fori_loop(0, CHUNK, row_body, 0)

    start_gather(0, 0)

    def loop_body(it, carry):
        g0 = it * 2
        for b in range(2):
            g = g0 + b
            other = 1 - b
            wait_gather(g, b)
            pl.when(g > 0)(lambda: wait_scatter(g - 1, other))
            pl.when(g + 1 < CHUNKS_PER_WORKER)(
                lambda: start_gather(g + 1, other))
            add_pos(g, b)
            start_scatter(g, b)
        return carry

    lax.fori_loop(0, CHUNKS_PER_WORKER // 2, loop_body, 0)
    wait_scatter(CHUNKS_PER_WORKER - 1, 1)


@jax.jit
def kernel(x, token_table, pos_table):
    x2 = x.reshape(NUM_WORKERS * CHUNKS_PER_WORKER, CHUNK)
    mesh = plsc.VectorSubcoreMesh(core_axis_name="c", subcore_axis_name="s")
    out = pl.kernel(
        _sc_body,
        out_type=jax.ShapeDtypeStruct((TOTAL_ROWS, EMB), jnp.float32),
        mesh=mesh,
        scratch_types=[
            pltpu.VMEM((CHUNKS_PER_WORKER, CHUNK), jnp.int32),
            pltpu.VMEM((MAXLEN + CHUNK, EMB), jnp.float32),
            pltpu.VMEM((CHUNK, EMB), jnp.float32),
            pltpu.VMEM((CHUNK, EMB), jnp.float32),
            pltpu.SemaphoreType.DMA,
            pltpu.SemaphoreType.DMA,
            pltpu.SemaphoreType.DMA,
            pltpu.SemaphoreType.DMA,
        ],
        compiler_params=pltpu.CompilerParams(use_tc_tiling_on_sc=False),
    )(x2, token_table, pos_table)
    return out.reshape(BATCH, MAXLEN, EMB)


# speedup vs baseline: 3.7599x; 1.4716x over previous
import functools

import jax
import jax.numpy as jnp
from jax import lax
from jax.experimental import pallas as pl
from jax.experimental.pallas import tpu as pltpu
from jax.experimental.pallas import tpu_sc as plsc

BATCH = 4096
MAXLEN = 200
EMB = 64

NUM_CORES = 2
NUM_SUBCORES = 16
NUM_WORKERS = NUM_CORES * NUM_SUBCORES
TOTAL_ROWS = BATCH * MAXLEN
ROWS_PER_WORKER = TOTAL_ROWS // NUM_WORKERS
CHUNK = 128
CHUNKS_PER_WORKER = ROWS_PER_WORKER // CHUNK
LANES = 16
VECS_PER_ROW = EMB // LANES


def _sc_body(x2_hbm, tab_hbm, pos_hbm, out_hbm, idx_v, posrep_v,
             rows_a, rows_b, sg_a, sg_b, ss_a, ss_b):
    c = lax.axis_index("c")
    s = lax.axis_index("s")
    wid = s * NUM_CORES + c

    rows = (rows_a, rows_b)
    sg = (sg_a, sg_b)
    ss = (ss_a, ss_b)

    pltpu.sync_copy(pos_hbm, posrep_v.at[pl.ds(0, MAXLEN)])
    pltpu.sync_copy(pos_hbm.at[pl.ds(0, CHUNK)],
                    posrep_v.at[pl.ds(MAXLEN, CHUNK)])

    pltpu.sync_copy(x2_hbm.at[pl.ds(wid * CHUNKS_PER_WORKER, CHUNKS_PER_WORKER)],
                    idx_v)

    def out_slice(g):
        return out_hbm.at[pl.ds(wid * ROWS_PER_WORKER + g * CHUNK, CHUNK)]

    def start_gather(g, b):
        pltpu.async_copy(tab_hbm.at[idx_v.at[g]], rows[b], sg[b])

    def wait_gather(g, b):
        pltpu.make_async_copy(tab_hbm.at[idx_v.at[g]], rows[b], sg[b]).wait()

    def start_scatter(g, b):
        pltpu.async_copy(rows[b], out_slice(g), ss[b])

    def wait_scatter(g, b):
        pltpu.make_async_copy(rows[b], out_slice(g), ss[b]).wait()

    def add_pos(g, b):
        buf = rows[b]
        phase = lax.rem(g * CHUNK, MAXLEN)

        @plsc.parallel_loop(0, CHUNK, step=1, unroll=8)
        def _row(i):
            for j in range(VECS_PER_ROW):
                sl = pl.ds(j * LANES, LANES)
                buf[i, sl] = buf[i, sl] + posrep_v[phase + i, sl]

    start_gather(0, 0)

    def loop_body(it, carry):
        g0 = it * 2
        for b in range(2):
            g = g0 + b
            other = 1 - b
            wait_gather(g, b)
            pl.when(g > 0)(lambda: wait_scatter(g - 1, other))
            pl.when(g + 1 < CHUNKS_PER_WORKER)(
                lambda: start_gather(g + 1, other))
            add_pos(g, b)
            start_scatter(g, b)
        return carry

    lax.fori_loop(0, CHUNKS_PER_WORKER // 2, loop_body, 0)
    wait_scatter(CHUNKS_PER_WORKER - 1, 1)


@jax.jit
def kernel(x, token_table, pos_table):
    x2 = x.reshape(NUM_WORKERS * CHUNKS_PER_WORKER, CHUNK)
    mesh = plsc.VectorSubcoreMesh(core_axis_name="c", subcore_axis_name="s")
    out = pl.kernel(
        _sc_body,
        out_type=jax.ShapeDtypeStruct((TOTAL_ROWS, EMB), jnp.float32),
        mesh=mesh,
        scratch_types=[
            pltpu.VMEM((CHUNKS_PER_WORKER, CHUNK), jnp.int32),
            pltpu.VMEM((MAXLEN + CHUNK, EMB), jnp.float32),
            pltpu.VMEM((CHUNK, EMB), jnp.float32),
            pltpu.VMEM((CHUNK, EMB), jnp.float32),
            pltpu.SemaphoreType.DMA,
            pltpu.SemaphoreType.DMA,
            pltpu.SemaphoreType.DMA,
            pltpu.SemaphoreType.DMA,
        ],
        compiler_params=pltpu.CompilerParams(use_tc_tiling_on_sc=False),
    )(x2, token_table, pos_table)
    return out.reshape(BATCH, MAXLEN, EMB)


# speedup vs baseline: 4.5821x; 1.2187x over previous
import functools

import jax
import jax.numpy as jnp
from jax import lax
from jax.experimental import pallas as pl
from jax.experimental.pallas import tpu as pltpu
from jax.experimental.pallas import tpu_sc as plsc

BATCH = 4096
MAXLEN = 200
EMB = 64
PADE = 128

NUM_CORES = 2
NUM_SUBCORES = 16
NUM_WORKERS = NUM_CORES * NUM_SUBCORES
ROWS_PER_WORKER = BATCH // NUM_WORKERS
G1 = 128
G2 = MAXLEN - G1
LANES = 16
VECS_PER_ROW = EMB // LANES


def _sc_body(x_hbm, tab_hbm, pos_hbm, out_hbm, pos_v,
             idx_a, idx_b, ga, gb, oa, ob, si_a, si_b, sg_a, sg_b,
             ss_a, ss_b):
    c = lax.axis_index("c")
    s = lax.axis_index("s")
    wid = s * NUM_CORES + c
    bstart = wid * ROWS_PER_WORKER

    idx = (idx_a, idx_b)
    gbuf = (ga, gb)
    obuf = (oa, ob)
    si = (si_a, si_b)
    sg = (sg_a, sg_b)
    ss = (ss_a, ss_b)

    pltpu.sync_copy(pos_hbm, pos_v)

    def start_idx(g, p):
        pltpu.async_copy(x_hbm.at[bstart + g], idx[p], si[p])

    def wait_idx(g, p):
        pltpu.make_async_copy(x_hbm.at[bstart + g], idx[p], si[p]).wait()

    def start_gather(p):
        pltpu.async_copy(tab_hbm.at[idx[p].at[pl.ds(0, G1)]],
                         gbuf[p].at[pl.ds(0, G1)], sg[p])
        pltpu.async_copy(tab_hbm.at[idx[p].at[pl.ds(G1, G2)]],
                         gbuf[p].at[pl.ds(G1, G2)], sg[p])

    def wait_gather(p):
        pltpu.make_async_copy(tab_hbm.at[idx[p].at[pl.ds(0, G1)]],
                              gbuf[p].at[pl.ds(0, G1)], sg[p]).wait()
        pltpu.make_async_copy(tab_hbm.at[idx[p].at[pl.ds(G1, G2)]],
                              gbuf[p].at[pl.ds(G1, G2)], sg[p]).wait()

    def start_scatter(g, p):
        pltpu.async_copy(obuf[p], out_hbm.at[bstart + g], ss[p])

    def wait_scatter(g, p):
        pltpu.make_async_copy(obuf[p], out_hbm.at[bstart + g], ss[p]).wait()

    def add_pos(p):
        src = gbuf[p]
        dst = obuf[p]

        @plsc.parallel_loop(0, MAXLEN, step=1, unroll=8)
        def _row(i):
            for j in range(VECS_PER_ROW):
                sl = pl.ds(j * LANES, LANES)
                dst[i, sl] = src[i, sl] + pos_v[i, sl]

    pltpu.sync_copy(x_hbm.at[bstart], idx_a)
    start_idx(1, 1)
    start_gather(0)

    def loop_body(it, carry):
        g0 = it * 2
        for b in range(2):
            g = g0 + b
            p = b
            q = 1 - b
            wait_gather(p)
            pl.when(g + 1 < ROWS_PER_WORKER)(lambda: wait_idx(g + 1, q))
            pl.when(g + 1 < ROWS_PER_WORKER)(lambda: start_gather(q))
            pl.when(g + 2 < ROWS_PER_WORKER)(lambda: start_idx(g + 2, p))
            pl.when(g > 1)(lambda: wait_scatter(g - 2, p))
            add_pos(p)
            start_scatter(g, p)
        return carry

    lax.fori_loop(0, ROWS_PER_WORKER // 2, loop_body, 0)
    wait_scatter(ROWS_PER_WORKER - 2, 0)
    wait_scatter(ROWS_PER_WORKER - 1, 1)


@jax.jit
def kernel(x, token_table, pos_table):
    tab_pad = jnp.pad(token_table, ((0, 0), (0, PADE - EMB)))
    mesh = plsc.VectorSubcoreMesh(core_axis_name="c", subcore_axis_name="s")
    return pl.kernel(
        _sc_body,
        out_type=jax.ShapeDtypeStruct((BATCH, MAXLEN, EMB), jnp.float32),
        mesh=mesh,
        scratch_types=[
            pltpu.VMEM((MAXLEN, EMB), jnp.float32),
            pltpu.VMEM((MAXLEN,), jnp.int32),
            pltpu.VMEM((MAXLEN,), jnp.int32),
            pltpu.VMEM((MAXLEN, PADE), jnp.float32),
            pltpu.VMEM((MAXLEN, PADE), jnp.float32),
            pltpu.VMEM((MAXLEN, EMB), jnp.float32),
            pltpu.VMEM((MAXLEN, EMB), jnp.float32),
            pltpu.SemaphoreType.DMA,
            pltpu.SemaphoreType.DMA,
            pltpu.SemaphoreType.DMA,
            pltpu.SemaphoreType.DMA,
            pltpu.SemaphoreType.DMA,
            pltpu.SemaphoreType.DMA,
        ],
    )(x, tab_pad, pos_table)
